# baseline (speedup 1.0000x reference)
import numpy as np
import jax
import jax.numpy as jnp
from jax import lax
from jax.experimental import pallas as pl
from jax.experimental.pallas import tpu as pltpu
from jax.experimental.pallas import tpu_sc as plsc

_BOUND = 1.6
_T = 2 ** 19
_MASK = _T - 1
_LROWS = 2 ** 20 // 8
_N = 65536
_NC = 2
_NS = 16
_NW = _NC * _NS
_CHUNK = _N // _NW
_NG = _CHUNK // 16
_SB = 112

_P = [1, -1640531535, 805459861, -620313867]


def _res_table(base, desired, levels):
    base = np.asarray(base, dtype=np.float64)
    desired = np.asarray(desired, dtype=np.float64)
    scale = np.exp((np.log(desired) - np.log(base)) / max(levels - 1, 1))
    lv = np.arange(levels, dtype=np.float64)[:, None]
    res = np.floor(base[None, :] * (scale[None, :] ** lv)).astype(np.int64)
    return np.maximum(res, 2)


_RES3 = _res_table([16.0] * 3, [2048.0] * 3, 16)
_RES4 = _res_table([8.0] * 4, [32.0, 32.0, 16.0, 16.0], 32)

_P4 = [int(np.prod(_RES4[l])) for l in range(32)]
_NSB4 = [-(-p // _SB) for p in _P4]
_O4 = []
_G4ROWS = []
for _c in range(4):
    _offs = np.cumsum([0] + [4 * _SB * n for n in _NSB4[8 * _c:8 * _c + 8]])
    _O4.extend(_offs[:8].tolist())
    _G4ROWS.append(int(_offs[8]) // 8 + 1)


def _corner_hashes_weights(xn, res_row, want_hash=True):
    hs, ws = [], []
    for d in range(len(xn)):
        fr = jnp.float32(int(res_row[d]) - 1)
        ci = jnp.int32(int(res_row[d]) - 1)
        pos = xn[d] * fr
        c0 = pos.astype(jnp.int32)
        w = pos - c0.astype(jnp.float32)
        c1 = jnp.minimum(c0 + 1, ci)
        if want_hash and _P[d] != 1:
            h0, h1 = c0 * jnp.int32(_P[d]), c1 * jnp.int32(_P[d])
        else:
            h0, h1 = c0, c1
        hs.append((h0, h1))
        ws.append((jnp.float32(1.0) - w, w))
    return hs, ws


def _rows_of(e, l):
    row = lax.shift_right_logical(e, 2) + jnp.int32(l * _LROWS)
    return row, lax.shift_left(e & jnp.int32(3), 1)


def _emit_level3(l, xn, idx_ref, lob_ref, w_ref):
    hs, ws = _corner_hashes_weights(xn, _RES3[l])
    hyz = [[hs[1][by] ^ hs[2][bz] for bz in (0, 1)] for by in (0, 1)]
    wxy = [[ws[0][bx] * ws[1][by] for by in (0, 1)] for bx in (0, 1)]
    for k in range(8):
        bx, by, bz = (k >> 2) & 1, (k >> 1) & 1, k & 1
        e = (hs[0][bx] ^ hyz[by][bz]) & jnp.int32(_MASK)
        row0, off = _rows_of(e, l)
        idx_ref[l, pl.ds(k * 16, 16)] = row0
        lob_ref[l, k] = off
        w_ref[l, k] = wxy[bx][by] * ws[2][bz]


def _emit_level4(lrel, lglob, xn, idx_ref, lob_ref, w_ref):
    res = _RES4[lglob]
    rx, ry, rz = int(res[0]), int(res[1]), int(res[2])
    my, mz, mt = rx, rx * ry, rx * ry * rz
    hs, ws = _corner_hashes_weights(xn, res, want_hash=False)
    sy = (hs[1][0] * jnp.int32(my), hs[1][1] * jnp.int32(my))
    sz = (hs[2][0] * jnp.int32(mz), hs[2][1] * jnp.int32(mz))
    st = (hs[3][0] * jnp.int32(mt), hs[3][1] * jnp.int32(mt))
    szt = [[sz[a] + st[b] for b in (0, 1)] for a in (0, 1)]
    wxy = [[ws[0][bx] * ws[1][by] for by in (0, 1)] for bx in (0, 1)]
    wxyz = [[[wxy[bx][by] * ws[2][bz] for bz in (0, 1)] for by in (0, 1)]
            for bx in (0, 1)]
    o4l = jnp.int32(_O4[lglob])
    for p in range(8):
        by, bz, bt = (p >> 2) & 1, (p >> 1) & 1, p & 1
        pid = hs[0][0] + (sy[by] + szt[bz][bt])
        word = pid * jnp.int32(4) + o4l
        idx_ref[lrel, pl.ds(p * 16, 16)] = lax.shift_right_logical(word, 3)
        lob_ref[lrel, p] = word & jnp.int32(7)
        w_ref[lrel, 2 * p] = wxyz[0][by][bz] * ws[3][bt]
        w_ref[lrel, 2 * p + 1] = wxyz[1][by][bz] * ws[3][bt]


def _combine3(rows_ref, rb, lob_ref, w_ref, l, lane):
    acc0 = acc1 = None
    for k in range(8):
        rk = lane + (rb + k * 16)
        off = lob_ref[l, k]
        f0 = plsc.load_gather(rows_ref, [rk, off])
        f1 = plsc.load_gather(rows_ref, [rk, off + 1])
        wk = w_ref[l, k]
        if acc0 is None:
            acc0, acc1 = f0 * wk, f1 * wk
        else:
            acc0, acc1 = acc0 + f0 * wk, acc1 + f1 * wk
    return acc0, acc1


def _combine4(rows_ref, rb, lob_ref, w_ref, lrel, lane):
    acc0 = acc1 = None
    for p in range(8):
        rk = lane + (rb + p * 16)
        off = lob_ref[lrel, p]
        f00 = plsc.load_gather(rows_ref, [rk, off])
        f10 = plsc.load_gather(rows_ref, [rk, off + 1])
        f01 = plsc.load_gather(rows_ref, [rk, off + 2])
        f11 = plsc.load_gather(rows_ref, [rk, off + 3])
        w0 = w_ref[lrel, 2 * p]
        w1 = w_ref[lrel, 2 * p + 1]
        if acc0 is None:
            acc0 = f00 * w0 + f01 * w1
            acc1 = f10 * w0 + f11 * w1
        else:
            acc0 = acc0 + f00 * w0 + f01 * w1
            acc1 = acc1 + f10 * w0 + f11 * w1
    return acc0, acc1


def _build4_body(chunk, t4, grid2, idxb, offb, land, stage, sem):
    wid = lax.axis_index("s") * _NC + lax.axis_index("c")
    lane = lax.iota(jnp.int32, 16)
    for lglob in range(8 * chunk, 8 * chunk + 8):
        res = _RES4[lglob]
        rx, ry, rz = int(res[0]), int(res[1]), int(res[2])
        pmax = jnp.int32(_P4[lglob] - 1)
        nsb = _NSB4[lglob]
        base_row = _O4[lglob] // 8

        def body_sb(i, carry, lglob=lglob, rx=rx, ry=ry, rz=rz,
                    pmax=pmax, nsb=nsb, base_row=base_row):
            sb = i * _NW + wid

            @pl.when(sb < nsb)
            def _():
                s = sb * _SB
                def idiv(v, d):
                    return ((v.astype(jnp.float32) + jnp.float32(0.5))
                            * jnp.float32(1.0 / d)).astype(jnp.int32)

                for j in range(8):
                    v = jnp.minimum(s + (j * 16) + lane, pmax)
                    q = idiv(v, rx)
                    cx = v - q * jnp.int32(rx)
                    q2 = idiv(q, ry)
                    cy = q - q2 * jnp.int32(ry)
                    ct = idiv(q2, rz)
                    cz = q2 - ct * jnp.int32(rz)
                    e = (cx ^ (cy * jnp.int32(_P[1]))
                         ^ (cz * jnp.int32(_P[2]))
                         ^ (ct * jnp.int32(_P[3]))) & jnp.int32(_MASK)
                    row0, off = _rows_of(e, lglob)
                    idxb[0, pl.ds(j * 16, 16)] = row0
                    offb[j] = off
                cp0 = pltpu.async_copy(t4.at[idxb.at[0]],
                                       land.at[pl.ds(0, 128)], sem)
                cp0.wait()
                for j in range(8):
                    off = offb[j]
                    f0 = plsc.load_gather(land, [lane + j * 16, off])
                    f1 = plsc.load_gather(land, [lane + j * 16, off + 1])
                    rel = lane * jnp.int32(4) + jnp.int32(j * 64)
                    hi = lax.shift_right_logical(rel, 3)
                    lo = rel & jnp.int32(7)
                    if j < 7:
                        plsc.store_scatter(stage, [hi, lo], f0)
                        plsc.store_scatter(stage, [hi, lo + 1], f1)
                    reld = rel - 2
                    hid = lax.shift_right_logical(reld, 3)
                    lod = reld & jnp.int32(7)
                    if j == 0:
                        m = lane > 0
                        plsc.store_scatter(stage, [hid, lod], f0, mask=m)
                        plsc.store_scatter(stage, [hid, lod + 1], f1, mask=m)
                    elif j < 7:
                        plsc.store_scatter(stage, [hid, lod], f0)
                        plsc.store_scatter(stage, [hid, lod + 1], f1)
                    else:
                        m = lane == 0
                        plsc.store_scatter(stage, [hid, lod], f0, mask=m)
                        plsc.store_scatter(stage, [hid, lod + 1], f1, mask=m)
                pltpu.sync_copy(
                    stage, grid2.at[pl.ds(base_row + sb * (_SB // 2), _SB // 2)])
            return carry

        lax.fori_loop(0, -(-nsb // _NW), body_sb, 0)


def _dash4d_body(xw, yw, zw, tw, t3, g40, g41, g42, g43, out3, out4a, out4b,
                 cx, cy, cz, ct, idx3, lob3, w3,
                 idx4, lob4, w4, rows, ob3, ob4, sem):
    grids = (g40, g41, g42, g43)
    wid = lax.axis_index("s") * _NC + lax.axis_index("c")
    base = wid * _CHUNK
    pltpu.sync_copy(xw.at[pl.ds(base, _CHUNK)], cx)
    pltpu.sync_copy(yw.at[pl.ds(base, _CHUNK)], cy)
    pltpu.sync_copy(zw.at[pl.ds(base, _CHUNK)], cz)
    pltpu.sync_copy(tw.at[pl.ds(base, _CHUNK)], ct)

    lane = lax.iota(jnp.int32, 16)
    den = jnp.float32(2.0 * _BOUND)

    def _coords(g, with_t):
        sl = pl.ds(g * 16, 16)
        vals = [cx[sl], cy[sl], cz[sl]] + ([ct[sl]] if with_t else [])
        return [jnp.minimum(jnp.maximum((v + jnp.float32(_BOUND)) / den,
                                        jnp.float32(0.0)), jnp.float32(1.0))
                for v in vals]

    def body3(g, carry):
        xn = _coords(g, False)
        cps = []
        for l in range(16):
            _emit_level3(l, xn, idx3, lob3, w3)
            cps.append(pltpu.async_copy(
                t3.at[idx3.at[l]], rows.at[pl.ds(l * 128, 128)], sem))
        for l in range(16):
            cps[l].wait()
            a0, a1 = _combine3(rows, l * 128, lob3, w3, l, lane)
            plsc.store_scatter(ob3, [lane, jnp.full((16,), 2 * l, jnp.int32)], a0)
            plsc.store_scatter(ob3, [lane, jnp.full((16,), 2 * l + 1, jnp.int32)], a1)
        pltpu.sync_copy(ob3, out3.at[pl.ds(base + g * 16, 16)])
        return carry

    lax.fori_loop(0, _NG, body3, 0)

    def _body4(g, lo, out_ref):
        xn = _coords(g, True)
        cps = []
        for lrel in range(16):
            _emit_level4(lrel, lo + lrel, xn, idx4, lob4, w4)
            gref = grids[(lo + lrel) // 8]
            cps.append(pltpu.async_copy(
                gref.at[idx4.at[lrel]], rows.at[pl.ds(lrel * 128, 128)], sem))
        for lrel in range(16):
            cps[lrel].wait()
            a0, a1 = _combine4(rows, lrel * 128, lob4, w4, lrel, lane)
            plsc.store_scatter(ob4, [lane, jnp.full((16,), 2 * lrel, jnp.int32)], a0)
            plsc.store_scatter(ob4, [lane, jnp.full((16,), 2 * lrel + 1, jnp.int32)], a1)
        pltpu.sync_copy(ob4, out_ref.at[pl.ds(base + g * 16, 16)])
        return 0

    lax.fori_loop(0, _NG, lambda g, c: _body4(g, 0, out4a), 0)
    lax.fori_loop(0, _NG, lambda g, c: _body4(g, 16, out4b), 0)


def kernel(xyzt, table3, table4):
    f32 = jnp.float32
    xw = xyzt[:, 0]
    yw = xyzt[:, 1]
    zw = xyzt[:, 2]
    tw = xyzt[:, 3]
    t3 = table3.reshape(16 * _LROWS, 8)
    t4 = table4.reshape(32 * _LROWS, 8)

    mesh = plsc.VectorSubcoreMesh(core_axis_name="c", subcore_axis_name="s",
                                  num_cores=_NC, num_subcores=_NS)
    cparams = pltpu.CompilerParams(
        needs_layout_passes=False, use_tc_tiling_on_sc=False)

    import functools
    grids = []
    for c in range(4):
        build = pl.kernel(
            functools.partial(_build4_body, c),
            out_type=(jax.ShapeDtypeStruct((_G4ROWS[c], 8), f32),),
            mesh=mesh,
            compiler_params=cparams,
            scratch_types=[
                pltpu.VMEM((2, 128), jnp.int32),
                pltpu.VMEM((8, 16), jnp.int32),
                pltpu.VMEM((256, 8), f32),
                pltpu.VMEM((_SB // 2, 8), f32),
                pltpu.SemaphoreType.DMA,
            ],
            name=f"build4_{c}",
        )
        (g,) = build(t4)
        grids.append(g)

    fn = pl.kernel(
        _dash4d_body,
        out_type=(
            jax.ShapeDtypeStruct((_N, 32), f32),
            jax.ShapeDtypeStruct((_N, 32), f32),
            jax.ShapeDtypeStruct((_N, 32), f32),
        ),
        mesh=mesh,
        compiler_params=cparams,
        scratch_types=[
            pltpu.VMEM((_CHUNK,), f32),
            pltpu.VMEM((_CHUNK,), f32),
            pltpu.VMEM((_CHUNK,), f32),
            pltpu.VMEM((_CHUNK,), f32),
            pltpu.VMEM((16, 128), jnp.int32),
            pltpu.VMEM((16, 8, 16), jnp.int32),
            pltpu.VMEM((16, 8, 16), f32),
            pltpu.VMEM((16, 128), jnp.int32),
            pltpu.VMEM((16, 8, 16), jnp.int32),
            pltpu.VMEM((16, 16, 16), f32),
            pltpu.VMEM((32 * 128, 8), f32),
            pltpu.VMEM((16, 32), f32),
            pltpu.VMEM((16, 32), f32),
            pltpu.SemaphoreType.DMA,
        ],
    )
    out3, out4a, out4b = fn(xw, yw, zw, tw, t3, *grids)
    return out3, jnp.concatenate([out4a, out4b], axis=-1)


# speedup vs baseline: 11.5065x; 11.5065x over previous
import numpy as np
import jax
import jax.numpy as jnp
from jax import lax
from jax.experimental import pallas as pl
from jax.experimental.pallas import tpu as pltpu
from jax.experimental.pallas import tpu_sc as plsc

_BOUND = 1.6
_T = 2 ** 19
_MASK = _T - 1
_LROWS = 2 ** 20 // 8
_N = 65536
_NC = 2
_NS = 16
_NW = _NC * _NS
_CHUNK = _N // _NW
_NG = _CHUNK // 16
_SB = 112

_P = [1, -1640531535, 805459861, -620313867]


def _res_table(base, desired, levels):
    base = np.asarray(base, dtype=np.float64)
    desired = np.asarray(desired, dtype=np.float64)
    scale = np.exp((np.log(desired) - np.log(base)) / max(levels - 1, 1))
    lv = np.arange(levels, dtype=np.float64)[:, None]
    res = np.floor(base[None, :] * (scale[None, :] ** lv)).astype(np.int64)
    return np.maximum(res, 2)


_RES3 = _res_table([16.0] * 3, [2048.0] * 3, 16)
_RES4 = _res_table([8.0] * 4, [32.0, 32.0, 16.0, 16.0], 32)

_P4 = [int(np.prod(_RES4[l])) for l in range(32)]
_NSB4 = [-(-p // _SB) for p in _P4]
_O4 = []
_G4ROWS = []
for _c in range(4):
    _offs = np.cumsum([0] + [4 * _SB * n for n in _NSB4[8 * _c:8 * _c + 8]])
    _O4.extend(_offs[:8].tolist())
    _G4ROWS.append(int(_offs[8]) // 8 + 1)


def _corner_hashes_weights(xn, res_row, want_hash=True):
    hs, ws = [], []
    for d in range(len(xn)):
        fr = jnp.float32(int(res_row[d]) - 1)
        ci = jnp.int32(int(res_row[d]) - 1)
        pos = xn[d] * fr
        c0 = pos.astype(jnp.int32)
        w = pos - c0.astype(jnp.float32)
        c1 = jnp.minimum(c0 + 1, ci)
        if want_hash and _P[d] != 1:
            h0, h1 = c0 * jnp.int32(_P[d]), c1 * jnp.int32(_P[d])
        else:
            h0, h1 = c0, c1
        hs.append((h0, h1))
        ws.append((jnp.float32(1.0) - w, w))
    return hs, ws


def _rows_of(e, l):
    row = lax.shift_right_logical(e, 2) + jnp.int32(l * _LROWS)
    return row, lax.shift_left(e & jnp.int32(3), 1)


def _rows_of_nat(e, l):
    r = lax.shift_right_logical(e, 3)
    row0 = r + (r & jnp.int32(~15)) + jnp.int32(l * _LROWS)
    return row0, e & jnp.int32(7)


def _ileave_body(t, ti, land, stage):
    wid = lax.axis_index("s") * _NC + lax.axis_index("c")
    lane = lax.iota(jnp.int32, 16)
    l3 = lax.shift_right_logical(lane, 3)
    l2 = lax.shift_right_logical(lane, 2)
    lw = lane & jnp.int32(7)
    sw = lax.shift_left(lane & jnp.int32(3), 1)
    nrw = t.shape[0] // _NW

    def outer(i, c):
        base = wid * nrw + i * 2048
        pltpu.sync_copy(t.at[pl.ds(base, 2048)], land)

        def inner(k, c2):
            kb = k * jnp.int32(256)
            for bb in range(8):
                for j in range(8):
                    g0r = kb + jnp.int32(bb * 32 + j * 2) + l3
                    f0 = plsc.load_gather(land, [g0r, lw])
                    f1 = plsc.load_gather(land, [g0r + jnp.int32(16), lw])
                    sr = kb + jnp.int32(bb * 32 + j * 4) + l2
                    plsc.store_scatter(stage, [sr, sw], f0)
                    plsc.store_scatter(stage, [sr, sw + 1], f1)
            return c2

        lax.fori_loop(0, 8, inner, 0)
        pltpu.sync_copy(stage, ti.at[pl.ds(base, 2048)])
        return c

    lax.fori_loop(0, nrw // 2048, outer, 0)


def _emit_level3(l, xn, idx_ref, lob_ref, w_ref):
    hs, ws = _corner_hashes_weights(xn, _RES3[l])
    hyz = [[hs[1][by] ^ hs[2][bz] for bz in (0, 1)] for by in (0, 1)]
    wxy = [[ws[0][bx] * ws[1][by] for by in (0, 1)] for bx in (0, 1)]
    for k in range(8):
        bx, by, bz = (k >> 2) & 1, (k >> 1) & 1, k & 1
        e = (hs[0][bx] ^ hyz[by][bz]) & jnp.int32(_MASK)
        row0, off = _rows_of(e, l)
        idx_ref[l, pl.ds(k * 16, 16)] = row0
        lob_ref[l, k] = off
        w_ref[l, k] = wxy[bx][by] * ws[2][bz]


def _emit_level4(lrel, lglob, xn, idx_ref, lob_ref, w_ref):
    res = _RES4[lglob]
    rx, ry, rz = int(res[0]), int(res[1]), int(res[2])
    my, mz, mt = rx, rx * ry, rx * ry * rz
    hs, ws = _corner_hashes_weights(xn, res, want_hash=False)
    sy = (hs[1][0] * jnp.int32(my), hs[1][1] * jnp.int32(my))
    sz = (hs[2][0] * jnp.int32(mz), hs[2][1] * jnp.int32(mz))
    st = (hs[3][0] * jnp.int32(mt), hs[3][1] * jnp.int32(mt))
    szt = [[sz[a] + st[b] for b in (0, 1)] for a in (0, 1)]
    wxy = [[ws[0][bx] * ws[1][by] for by in (0, 1)] for bx in (0, 1)]
    wxyz = [[[wxy[bx][by] * ws[2][bz] for bz in (0, 1)] for by in (0, 1)]
            for bx in (0, 1)]
    o4l = jnp.int32(_O4[lglob])
    for p in range(8):
        by, bz, bt = (p >> 2) & 1, (p >> 1) & 1, p & 1
        pid = hs[0][0] + (sy[by] + szt[bz][bt])
        word = pid * jnp.int32(4) + o4l
        idx_ref[lrel, pl.ds(p * 16, 16)] = lax.shift_right_logical(word, 3)
        lob_ref[lrel, p] = word & jnp.int32(7)
        w_ref[lrel, 2 * p] = wxyz[0][by][bz] * ws[3][bt]
        w_ref[lrel, 2 * p + 1] = wxyz[1][by][bz] * ws[3][bt]


def _combine3(rows_ref, rb, lob_ref, w_ref, l, lane):
    acc0 = acc1 = None
    for k in range(8):
        rk = lane + (rb + k * 16)
        off = lob_ref[l, k]
        f0 = plsc.load_gather(rows_ref, [rk, off])
        f1 = plsc.load_gather(rows_ref, [rk, off + 1])
        wk = w_ref[l, k]
        if acc0 is None:
            acc0, acc1 = f0 * wk, f1 * wk
        else:
            acc0, acc1 = acc0 + f0 * wk, acc1 + f1 * wk
    return acc0, acc1


def _combine4(rows_ref, rb, lob_ref, w_ref, lrel, lane):
    acc0 = acc1 = None
    for p in range(8):
        rk = lane + (rb + p * 16)
        off = lob_ref[lrel, p]
        f00 = plsc.load_gather(rows_ref, [rk, off])
        f10 = plsc.load_gather(rows_ref, [rk, off + 1])
        f01 = plsc.load_gather(rows_ref, [rk, off + 2])
        f11 = plsc.load_gather(rows_ref, [rk, off + 3])
        w0 = w_ref[lrel, 2 * p]
        w1 = w_ref[lrel, 2 * p + 1]
        if acc0 is None:
            acc0 = f00 * w0 + f01 * w1
            acc1 = f10 * w0 + f11 * w1
        else:
            acc0 = acc0 + f00 * w0 + f01 * w1
            acc1 = acc1 + f10 * w0 + f11 * w1
    return acc0, acc1


def _build4_body(chunk, t4, grid2, idxb, offb, land, stage, sem):
    wid = lax.axis_index("s") * _NC + lax.axis_index("c")
    lane = lax.iota(jnp.int32, 16)
    for lglob in range(8 * chunk, 8 * chunk + 8):
        res = _RES4[lglob]
        rx, ry, rz = int(res[0]), int(res[1]), int(res[2])
        pmax = jnp.int32(_P4[lglob] - 1)
        nsb = _NSB4[lglob]
        base_row = _O4[lglob] // 8

        def body_sb(i, carry, lglob=lglob, rx=rx, ry=ry, rz=rz,
                    pmax=pmax, nsb=nsb, base_row=base_row):
            sb = i * _NW + wid

            @pl.when(sb < nsb)
            def _():
                s = sb * _SB
                def idiv(v, d):
                    return ((v.astype(jnp.float32) + jnp.float32(0.5))
                            * jnp.float32(1.0 / d)).astype(jnp.int32)

                for j in range(8):
                    v = jnp.minimum(s + (j * 16) + lane, pmax)
                    q = idiv(v, rx)
                    cx = v - q * jnp.int32(rx)
                    q2 = idiv(q, ry)
                    cy = q - q2 * jnp.int32(ry)
                    ct = idiv(q2, rz)
                    cz = q2 - ct * jnp.int32(rz)
                    e = (cx ^ (cy * jnp.int32(_P[1]))
                         ^ (cz * jnp.int32(_P[2]))
                         ^ (ct * jnp.int32(_P[3]))) & jnp.int32(_MASK)
                    row0, off = _rows_of_nat(e, lglob)
                    idxb[0, pl.ds(j * 16, 16)] = row0
                    idxb[1, pl.ds(j * 16, 16)] = row0 + 16
                    offb[j] = off
                cp0 = pltpu.async_copy(t4.at[idxb.at[0]],
                                       land.at[pl.ds(0, 128)], sem)
                cp1 = pltpu.async_copy(t4.at[idxb.at[1]],
                                       land.at[pl.ds(128, 128)], sem)
                cp0.wait()
                cp1.wait()
                for j in range(8):
                    off = offb[j]
                    f0 = plsc.load_gather(land, [lane + j * 16, off])
                    f1 = plsc.load_gather(land, [lane + 128 + j * 16, off])
                    rel = lane * jnp.int32(4) + jnp.int32(j * 64)
                    hi = lax.shift_right_logical(rel, 3)
                    lo = rel & jnp.int32(7)
                    if j < 7:
                        plsc.store_scatter(stage, [hi, lo], f0)
                        plsc.store_scatter(stage, [hi, lo + 1], f1)
                    reld = rel - 2
                    hid = lax.shift_right_logical(reld, 3)
                    lod = reld & jnp.int32(7)
                    if j == 0:
                        m = lane > 0
                        plsc.store_scatter(stage, [hid, lod], f0, mask=m)
                        plsc.store_scatter(stage, [hid, lod + 1], f1, mask=m)
                    elif j < 7:
                        plsc.store_scatter(stage, [hid, lod], f0)
                        plsc.store_scatter(stage, [hid, lod + 1], f1)
                    else:
                        m = lane == 0
                        plsc.store_scatter(stage, [hid, lod], f0, mask=m)
                        plsc.store_scatter(stage, [hid, lod + 1], f1, mask=m)
                pltpu.sync_copy(
                    stage, grid2.at[pl.ds(base_row + sb * (_SB // 2), _SB // 2)])
            return carry

        lax.fori_loop(0, -(-nsb // _NW), body_sb, 0)


def _dash4d_body(xw, yw, zw, tw, t3, g40, g41, g42, g43, out3, out4a, out4b,
                 cx, cy, cz, ct, idx3, lob3, w3,
                 idx4, lob4, w4, rows, ob3, ob4, sem):
    grids = (g40, g41, g42, g43)
    wid = lax.axis_index("s") * _NC + lax.axis_index("c")
    base = wid * _CHUNK
    pltpu.sync_copy(xw.at[pl.ds(base, _CHUNK)], cx)
    pltpu.sync_copy(yw.at[pl.ds(base, _CHUNK)], cy)
    pltpu.sync_copy(zw.at[pl.ds(base, _CHUNK)], cz)
    pltpu.sync_copy(tw.at[pl.ds(base, _CHUNK)], ct)

    lane = lax.iota(jnp.int32, 16)
    den = jnp.float32(2.0 * _BOUND)

    def _coords(g, with_t):
        sl = pl.ds(g * 16, 16)
        vals = [cx[sl], cy[sl], cz[sl]] + ([ct[sl]] if with_t else [])
        return [jnp.minimum(jnp.maximum((v + jnp.float32(_BOUND)) / den,
                                        jnp.float32(0.0)), jnp.float32(1.0))
                for v in vals]

    def body3(g, carry):
        xn = _coords(g, False)
        cps = []
        for l in range(16):
            _emit_level3(l, xn, idx3, lob3, w3)
            cps.append(pltpu.async_copy(
                t3.at[idx3.at[l]], rows.at[pl.ds(l * 128, 128)], sem))
        for l in range(16):
            cps[l].wait()
            a0, a1 = _combine3(rows, l * 128, lob3, w3, l, lane)
            plsc.store_scatter(ob3, [lane, jnp.full((16,), 2 * l, jnp.int32)], a0)
            plsc.store_scatter(ob3, [lane, jnp.full((16,), 2 * l + 1, jnp.int32)], a1)
        pltpu.sync_copy(ob3, out3.at[pl.ds(base + g * 16, 16)])
        return carry

    lax.fori_loop(0, _NG, body3, 0)

    def _body4(g, lo, out_ref):
        xn = _coords(g, True)
        cps = []
        for lrel in range(16):
            _emit_level4(lrel, lo + lrel, xn, idx4, lob4, w4)
            gref = grids[(lo + lrel) // 8]
            cps.append(pltpu.async_copy(
                gref.at[idx4.at[lrel]], rows.at[pl.ds(lrel * 128, 128)], sem))
        for lrel in range(16):
            cps[lrel].wait()
            a0, a1 = _combine4(rows, lrel * 128, lob4, w4, lrel, lane)
            plsc.store_scatter(ob4, [lane, jnp.full((16,), 2 * lrel, jnp.int32)], a0)
            plsc.store_scatter(ob4, [lane, jnp.full((16,), 2 * lrel + 1, jnp.int32)], a1)
        pltpu.sync_copy(ob4, out_ref.at[pl.ds(base + g * 16, 16)])
        return 0

    lax.fori_loop(0, _NG, lambda g, c: _body4(g, 0, out4a), 0)
    lax.fori_loop(0, _NG, lambda g, c: _body4(g, 16, out4b), 0)


def kernel(xyzt, table3, table4):
    f32 = jnp.float32
    xw = xyzt[:, 0]
    yw = xyzt[:, 1]
    zw = xyzt[:, 2]
    tw = xyzt[:, 3]
    t3 = (table3.reshape(16, _T // 128, 128, 2).transpose(0, 1, 3, 2)
          .reshape(16 * _LROWS, 8))
    t4 = (table4.reshape(32, _T // 128, 128, 2).transpose(0, 1, 3, 2)
          .reshape(32 * _LROWS, 8))

    mesh = plsc.VectorSubcoreMesh(core_axis_name="c", subcore_axis_name="s",
                                  num_cores=_NC, num_subcores=_NS)
    cparams = pltpu.CompilerParams(
        needs_layout_passes=False, use_tc_tiling_on_sc=False)

    import functools
    ileave = pl.kernel(
        _ileave_body,
        out_type=(jax.ShapeDtypeStruct((16 * _LROWS, 8), f32),),
        mesh=mesh,
        compiler_params=cparams,
        scratch_types=[
            pltpu.VMEM((2048, 8), f32),
            pltpu.VMEM((2048, 8), f32),
        ],
        name="ileave3",
    )
    (t3i,) = ileave(t3)
    grids = []
    for c in range(4):
        build = pl.kernel(
            functools.partial(_build4_body, c),
            out_type=(jax.ShapeDtypeStruct((_G4ROWS[c], 8), f32),),
            mesh=mesh,
            compiler_params=cparams,
            scratch_types=[
                pltpu.VMEM((2, 128), jnp.int32),
                pltpu.VMEM((8, 16), jnp.int32),
                pltpu.VMEM((256, 8), f32),
                pltpu.VMEM((_SB // 2, 8), f32),
                pltpu.SemaphoreType.DMA,
            ],
            name=f"build4_{c}",
        )
        (g,) = build(t4)
        grids.append(g)

    fn = pl.kernel(
        _dash4d_body,
        out_type=(
            jax.ShapeDtypeStruct((_N, 32), f32),
            jax.ShapeDtypeStruct((_N, 32), f32),
            jax.ShapeDtypeStruct((_N, 32), f32),
        ),
        mesh=mesh,
        compiler_params=cparams,
        scratch_types=[
            pltpu.VMEM((_CHUNK,), f32),
            pltpu.VMEM((_CHUNK,), f32),
            pltpu.VMEM((_CHUNK,), f32),
            pltpu.VMEM((_CHUNK,), f32),
            pltpu.VMEM((16, 128), jnp.int32),
            pltpu.VMEM((16, 8, 16), jnp.int32),
            pltpu.VMEM((16, 8, 16), f32),
            pltpu.VMEM((16, 128), jnp.int32),
            pltpu.VMEM((16, 8, 16), jnp.int32),
            pltpu.VMEM((16, 16, 16), f32),
            pltpu.VMEM((32 * 128, 8), f32),
            pltpu.VMEM((16, 32), f32),
            pltpu.VMEM((16, 32), f32),
            pltpu.SemaphoreType.DMA,
        ],
    )
    out3, out4a, out4b = fn(xw, yw, zw, tw, t3i, *grids)
    return out3, jnp.concatenate([out4a, out4b], axis=-1)


# speedup vs baseline: 13.0499x; 1.1341x over previous
import numpy as np
import jax
import jax.numpy as jnp
from jax import lax
from jax.experimental import pallas as pl
from jax.experimental.pallas import tpu as pltpu
from jax.experimental.pallas import tpu_sc as plsc

_BOUND = 1.6
_T = 2 ** 19
_MASK = _T - 1
_LROWS = 2 ** 20 // 8
_N = 65536
_NC = 2
_NS = 16
_NW = _NC * _NS
_CHUNK = _N // _NW
_NG = _CHUNK // 16
_SB = 112

_P = [1, -1640531535, 805459861, -620313867]


def _res_table(base, desired, levels):
    base = np.asarray(base, dtype=np.float64)
    desired = np.asarray(desired, dtype=np.float64)
    scale = np.exp((np.log(desired) - np.log(base)) / max(levels - 1, 1))
    lv = np.arange(levels, dtype=np.float64)[:, None]
    res = np.floor(base[None, :] * (scale[None, :] ** lv)).astype(np.int64)
    return np.maximum(res, 2)


_RES3 = _res_table([16.0] * 3, [2048.0] * 3, 16)
_RES4 = _res_table([8.0] * 4, [32.0, 32.0, 16.0, 16.0], 32)

_P4 = [int(np.prod(_RES4[l])) for l in range(32)]
_NSB4 = [-(-p // _SB) for p in _P4]
_O4 = []
_G4ROWS = []
for _c in range(4):
    _offs = np.cumsum([0] + [4 * _SB * n for n in _NSB4[8 * _c:8 * _c + 8]])
    _O4.extend(_offs[:8].tolist())
    _G4ROWS.append(int(_offs[8]) // 8 + 1)


def _corner_hashes_weights(xn, res_row, want_hash=True):
    hs, ws = [], []
    for d in range(len(xn)):
        fr = jnp.float32(int(res_row[d]) - 1)
        ci = jnp.int32(int(res_row[d]) - 1)
        pos = xn[d] * fr
        c0 = pos.astype(jnp.int32)
        w = pos - c0.astype(jnp.float32)
        c1 = jnp.minimum(c0 + 1, ci)
        if want_hash and _P[d] != 1:
            h0, h1 = c0 * jnp.int32(_P[d]), c1 * jnp.int32(_P[d])
        else:
            h0, h1 = c0, c1
        hs.append((h0, h1))
        ws.append((jnp.float32(1.0) - w, w))
    return hs, ws


def _rows_of(e, l):
    row = lax.shift_right_logical(e, 2) + jnp.int32(l * _LROWS)
    return row, lax.shift_left(e & jnp.int32(3), 1)


def _rows_of_nat(e, l):
    r = lax.shift_right_logical(e, 3)
    row0 = r + (r & jnp.int32(~15)) + jnp.int32(l * _LROWS)
    return row0, e & jnp.int32(7)


def _ileave_body(t, ti, land, stage):
    wid = lax.axis_index("s") * _NC + lax.axis_index("c")
    lane = lax.iota(jnp.int32, 16)
    l3 = lax.shift_right_logical(lane, 3)
    l2 = lax.shift_right_logical(lane, 2)
    lw = lane & jnp.int32(7)
    sw = lax.shift_left(lane & jnp.int32(3), 1)
    nrw = t.shape[0] // _NW

    def outer(i, c):
        base = wid * nrw + i * 2048
        pltpu.sync_copy(t.at[pl.ds(base, 2048)], land)

        def inner(k, c2):
            kb = k * jnp.int32(256)
            for bb in range(8):
                for j in range(8):
                    g0r = kb + jnp.int32(bb * 32 + j * 2) + l3
                    f0 = plsc.load_gather(land, [g0r, lw])
                    f1 = plsc.load_gather(land, [g0r + jnp.int32(16), lw])
                    sr = kb + jnp.int32(bb * 32 + j * 4) + l2
                    plsc.store_scatter(stage, [sr, sw], f0)
                    plsc.store_scatter(stage, [sr, sw + 1], f1)
            return c2

        lax.fori_loop(0, 8, inner, 0)
        pltpu.sync_copy(stage, ti.at[pl.ds(base, 2048)])
        return c

    lax.fori_loop(0, nrw // 2048, outer, 0)


def _emit_level3(l, xn, idx_ref, lob_ref, w_ref):
    hs, ws = _corner_hashes_weights(xn, _RES3[l])
    hyz = [[hs[1][by] ^ hs[2][bz] for bz in (0, 1)] for by in (0, 1)]
    wxy = [[ws[0][bx] * ws[1][by] for by in (0, 1)] for bx in (0, 1)]
    for k in range(8):
        bx, by, bz = (k >> 2) & 1, (k >> 1) & 1, k & 1
        e = (hs[0][bx] ^ hyz[by][bz]) & jnp.int32(_MASK)
        row0, off = _rows_of(e, l)
        idx_ref[l, pl.ds(k * 16, 16)] = row0
        lob_ref[l, k] = off
        w_ref[l, k] = wxy[bx][by] * ws[2][bz]


def _emit_level4(lrel, lglob, xn, idx_ref, lob_ref, w_ref):
    res = _RES4[lglob]
    rx, ry, rz = int(res[0]), int(res[1]), int(res[2])
    my, mz, mt = rx, rx * ry, rx * ry * rz
    hs, ws = _corner_hashes_weights(xn, res, want_hash=False)
    sy = (hs[1][0] * jnp.int32(my), hs[1][1] * jnp.int32(my))
    sz = (hs[2][0] * jnp.int32(mz), hs[2][1] * jnp.int32(mz))
    st = (hs[3][0] * jnp.int32(mt), hs[3][1] * jnp.int32(mt))
    szt = [[sz[a] + st[b] for b in (0, 1)] for a in (0, 1)]
    wxy = [[ws[0][bx] * ws[1][by] for by in (0, 1)] for bx in (0, 1)]
    wxyz = [[[wxy[bx][by] * ws[2][bz] for bz in (0, 1)] for by in (0, 1)]
            for bx in (0, 1)]
    o4l = jnp.int32(_O4[lglob])
    for p in range(8):
        by, bz, bt = (p >> 2) & 1, (p >> 1) & 1, p & 1
        pid = hs[0][0] + (sy[by] + szt[bz][bt])
        word = pid * jnp.int32(4) + o4l
        idx_ref[lrel, pl.ds(p * 16, 16)] = lax.shift_right_logical(word, 3)
        lob_ref[lrel, p] = word & jnp.int32(7)
        w_ref[lrel, 2 * p] = wxyz[0][by][bz] * ws[3][bt]
        w_ref[lrel, 2 * p + 1] = wxyz[1][by][bz] * ws[3][bt]


def _combine3(rows_ref, rb, lob_ref, w_ref, l, lane):
    acc0 = acc1 = None
    for k in range(8):
        rk = lane + (rb + k * 16)
        off = lob_ref[l, k]
        f0 = plsc.load_gather(rows_ref, [rk, off])
        f1 = plsc.load_gather(rows_ref, [rk, off + 1])
        wk = w_ref[l, k]
        if acc0 is None:
            acc0, acc1 = f0 * wk, f1 * wk
        else:
            acc0, acc1 = acc0 + f0 * wk, acc1 + f1 * wk
    return acc0, acc1


def _combine4(rows_ref, rb, lob_ref, w_ref, lrel, lane):
    acc0 = acc1 = None
    for p in range(8):
        rk = lane + (rb + p * 16)
        off = lob_ref[lrel, p]
        f00 = plsc.load_gather(rows_ref, [rk, off])
        f10 = plsc.load_gather(rows_ref, [rk, off + 1])
        f01 = plsc.load_gather(rows_ref, [rk, off + 2])
        f11 = plsc.load_gather(rows_ref, [rk, off + 3])
        w0 = w_ref[lrel, 2 * p]
        w1 = w_ref[lrel, 2 * p + 1]
        if acc0 is None:
            acc0 = f00 * w0 + f01 * w1
            acc1 = f10 * w0 + f11 * w1
        else:
            acc0 = acc0 + f00 * w0 + f01 * w1
            acc1 = acc1 + f10 * w0 + f11 * w1
    return acc0, acc1


def _build4_body(chunk, t4, grid2, idxb, offb, land, stage, sem):
    wid = lax.axis_index("s") * _NC + lax.axis_index("c")
    lane = lax.iota(jnp.int32, 16)
    for lglob in range(8 * chunk, 8 * chunk + 8):
        res = _RES4[lglob]
        rx, ry, rz = int(res[0]), int(res[1]), int(res[2])
        pmax = jnp.int32(_P4[lglob] - 1)
        nsb = _NSB4[lglob]
        base_row = _O4[lglob] // 8

        def body_sb(i, carry, lglob=lglob, rx=rx, ry=ry, rz=rz,
                    pmax=pmax, nsb=nsb, base_row=base_row):
            sb = i * _NW + wid

            @pl.when(sb < nsb)
            def _():
                s = sb * _SB
                def idiv(v, d):
                    return ((v.astype(jnp.float32) + jnp.float32(0.5))
                            * jnp.float32(1.0 / d)).astype(jnp.int32)

                for j in range(8):
                    v = jnp.minimum(s + (j * 16) + lane, pmax)
                    q = idiv(v, rx)
                    cx = v - q * jnp.int32(rx)
                    q2 = idiv(q, ry)
                    cy = q - q2 * jnp.int32(ry)
                    ct = idiv(q2, rz)
                    cz = q2 - ct * jnp.int32(rz)
                    e = (cx ^ (cy * jnp.int32(_P[1]))
                         ^ (cz * jnp.int32(_P[2]))
                         ^ (ct * jnp.int32(_P[3]))) & jnp.int32(_MASK)
                    row0, off = _rows_of_nat(e, lglob)
                    idxb[0, pl.ds(j * 16, 16)] = row0
                    idxb[1, pl.ds(j * 16, 16)] = row0 + 16
                    offb[j] = off
                cp0 = pltpu.async_copy(t4.at[idxb.at[0]],
                                       land.at[pl.ds(0, 128)], sem)
                cp1 = pltpu.async_copy(t4.at[idxb.at[1]],
                                       land.at[pl.ds(128, 128)], sem)
                cp0.wait()
                cp1.wait()
                for j in range(8):
                    off = offb[j]
                    f0 = plsc.load_gather(land, [lane + j * 16, off])
                    f1 = plsc.load_gather(land, [lane + 128 + j * 16, off])
                    rel = lane * jnp.int32(4) + jnp.int32(j * 64)
                    hi = lax.shift_right_logical(rel, 3)
                    lo = rel & jnp.int32(7)
                    if j < 7:
                        plsc.store_scatter(stage, [hi, lo], f0)
                        plsc.store_scatter(stage, [hi, lo + 1], f1)
                    reld = rel - 2
                    hid = lax.shift_right_logical(reld, 3)
                    lod = reld & jnp.int32(7)
                    if j == 0:
                        m = lane > 0
                        plsc.store_scatter(stage, [hid, lod], f0, mask=m)
                        plsc.store_scatter(stage, [hid, lod + 1], f1, mask=m)
                    elif j < 7:
                        plsc.store_scatter(stage, [hid, lod], f0)
                        plsc.store_scatter(stage, [hid, lod + 1], f1)
                    else:
                        m = lane == 0
                        plsc.store_scatter(stage, [hid, lod], f0, mask=m)
                        plsc.store_scatter(stage, [hid, lod + 1], f1, mask=m)
                pltpu.sync_copy(
                    stage, grid2.at[pl.ds(base_row + sb * (_SB // 2), _SB // 2)])
            return carry

        lax.fori_loop(0, -(-nsb // _NW), body_sb, 0)


def _dash4d_body(xw, yw, zw, tw, t3, g40, g41, g42, g43, out3, out4,
                 cx, cy, cz, ct, idx3, lob3, w3,
                 idx4, lob4, w4, rows, ob3, ob4, sem):
    grids = (g40, g41, g42, g43)
    wid = lax.axis_index("s") * _NC + lax.axis_index("c")
    base = wid * _CHUNK
    pltpu.sync_copy(xw.at[pl.ds(base, _CHUNK)], cx)
    pltpu.sync_copy(yw.at[pl.ds(base, _CHUNK)], cy)
    pltpu.sync_copy(zw.at[pl.ds(base, _CHUNK)], cz)
    pltpu.sync_copy(tw.at[pl.ds(base, _CHUNK)], ct)

    lane = lax.iota(jnp.int32, 16)
    den = jnp.float32(2.0 * _BOUND)

    def _coords(g, with_t):
        sl = pl.ds(g * 16, 16)
        vals = [cx[sl], cy[sl], cz[sl]] + ([ct[sl]] if with_t else [])
        return [jnp.minimum(jnp.maximum((v + jnp.float32(_BOUND)) / den,
                                        jnp.float32(0.0)), jnp.float32(1.0))
                for v in vals]

    def body(g, carry):
        xn = _coords(g, True)
        cps = []
        for l in range(16):
            _emit_level3(l, xn[:3], idx3, lob3, w3)
            cps.append(pltpu.async_copy(
                t3.at[idx3.at[l]], rows.at[pl.ds(l * 128, 128)], sem))
        for l in range(32):
            _emit_level4(l, l, xn, idx4, lob4, w4)
            gref = grids[l // 8]
            cps.append(pltpu.async_copy(
                gref.at[idx4.at[l]], rows.at[pl.ds(2048 + l * 128, 128)], sem))
        for l in range(16):
            cps[l].wait()
            a0, a1 = _combine3(rows, l * 128, lob3, w3, l, lane)
            plsc.store_scatter(ob3, [lane, jnp.full((16,), 2 * l, jnp.int32)], a0)
            plsc.store_scatter(ob3, [lane, jnp.full((16,), 2 * l + 1, jnp.int32)], a1)
        pltpu.sync_copy(ob3, out3.at[pl.ds(base + g * 16, 16)])
        for l in range(32):
            cps[16 + l].wait()
            a0, a1 = _combine4(rows, 2048 + l * 128, lob4, w4, l, lane)
            plsc.store_scatter(ob4, [lane, jnp.full((16,), 2 * l, jnp.int32)], a0)
            plsc.store_scatter(ob4, [lane, jnp.full((16,), 2 * l + 1, jnp.int32)], a1)
        pltpu.sync_copy(ob4, out4.at[pl.ds(base + g * 16, 16)])
        return carry

    lax.fori_loop(0, _NG, body, 0)


def kernel(xyzt, table3, table4):
    f32 = jnp.float32
    xw = xyzt[:, 0]
    yw = xyzt[:, 1]
    zw = xyzt[:, 2]
    tw = xyzt[:, 3]
    t3 = (table3.reshape(16, _T // 128, 128, 2).transpose(0, 1, 3, 2)
          .reshape(16 * _LROWS, 8))
    t4 = (table4.reshape(32, _T // 128, 128, 2).transpose(0, 1, 3, 2)
          .reshape(32 * _LROWS, 8))

    mesh = plsc.VectorSubcoreMesh(core_axis_name="c", subcore_axis_name="s",
                                  num_cores=_NC, num_subcores=_NS)
    cparams = pltpu.CompilerParams(
        needs_layout_passes=False, use_tc_tiling_on_sc=False)

    import functools
    ileave = pl.kernel(
        _ileave_body,
        out_type=(jax.ShapeDtypeStruct((16 * _LROWS, 8), f32),),
        mesh=mesh,
        compiler_params=cparams,
        scratch_types=[
            pltpu.VMEM((2048, 8), f32),
            pltpu.VMEM((2048, 8), f32),
        ],
        name="ileave3",
    )
    (t3i,) = ileave(t3)
    grids = []
    for c in range(4):
        build = pl.kernel(
            functools.partial(_build4_body, c),
            out_type=(jax.ShapeDtypeStruct((_G4ROWS[c], 8), f32),),
            mesh=mesh,
            compiler_params=cparams,
            scratch_types=[
                pltpu.VMEM((2, 128), jnp.int32),
                pltpu.VMEM((8, 16), jnp.int32),
                pltpu.VMEM((256, 8), f32),
                pltpu.VMEM((_SB // 2, 8), f32),
                pltpu.SemaphoreType.DMA,
            ],
            name=f"build4_{c}",
        )
        (g,) = build(t4)
        grids.append(g)

    fn = pl.kernel(
        _dash4d_body,
        out_type=(
            jax.ShapeDtypeStruct((_N, 32), f32),
            jax.ShapeDtypeStruct((_N, 64), f32),
        ),
        mesh=mesh,
        compiler_params=cparams,
        scratch_types=[
            pltpu.VMEM((_CHUNK,), f32),
            pltpu.VMEM((_CHUNK,), f32),
            pltpu.VMEM((_CHUNK,), f32),
            pltpu.VMEM((_CHUNK,), f32),
            pltpu.VMEM((16, 128), jnp.int32),
            pltpu.VMEM((16, 8, 16), jnp.int32),
            pltpu.VMEM((16, 8, 16), f32),
            pltpu.VMEM((32, 128), jnp.int32),
            pltpu.VMEM((32, 8, 16), jnp.int32),
            pltpu.VMEM((32, 16, 16), f32),
            pltpu.VMEM((48 * 128, 8), f32),
            pltpu.VMEM((16, 32), f32),
            pltpu.VMEM((16, 64), f32),
            pltpu.SemaphoreType.DMA,
        ],
    )
    out3, out4 = fn(xw, yw, zw, tw, t3i, *grids)
    return out3, out4


# speedup vs baseline: 13.0704x; 1.0016x over previous
import numpy as np
import jax
import jax.numpy as jnp
from jax import lax
from jax.experimental import pallas as pl
from jax.experimental.pallas import tpu as pltpu
from jax.experimental.pallas import tpu_sc as plsc

_BOUND = 1.6
_T = 2 ** 19
_MASK = _T - 1
_LROWS = 2 ** 20 // 8
_N = 65536
_NC = 2
_NS = 16
_NW = _NC * _NS
_CHUNK = _N // _NW
_NG = _CHUNK // 16
_SB = 112

_P = [1, -1640531535, 805459861, -620313867]


def _res_table(base, desired, levels):
    base = np.asarray(base, dtype=np.float64)
    desired = np.asarray(desired, dtype=np.float64)
    scale = np.exp((np.log(desired) - np.log(base)) / max(levels - 1, 1))
    lv = np.arange(levels, dtype=np.float64)[:, None]
    res = np.floor(base[None, :] * (scale[None, :] ** lv)).astype(np.int64)
    return np.maximum(res, 2)


_RES3 = _res_table([16.0] * 3, [2048.0] * 3, 16)
_RES4 = _res_table([8.0] * 4, [32.0, 32.0, 16.0, 16.0], 32)

_P4 = [int(np.prod(_RES4[l])) for l in range(32)]
_NSB4 = [-(-p // _SB) for p in _P4]
_O4 = []
_G4ROWS = []
for _c in range(4):
    _offs = np.cumsum([0] + [4 * _SB * n for n in _NSB4[8 * _c:8 * _c + 8]])
    _O4.extend(_offs[:8].tolist())
    _G4ROWS.append(int(_offs[8]) // 8 + 1)


def _corner_hashes_weights(xn, res_row, want_hash=True):
    hs, ws = [], []
    for d in range(len(xn)):
        fr = jnp.float32(int(res_row[d]) - 1)
        ci = jnp.int32(int(res_row[d]) - 1)
        pos = xn[d] * fr
        c0 = pos.astype(jnp.int32)
        w = pos - c0.astype(jnp.float32)
        c1 = jnp.minimum(c0 + 1, ci)
        if want_hash and _P[d] != 1:
            h0, h1 = c0 * jnp.int32(_P[d]), c1 * jnp.int32(_P[d])
        else:
            h0, h1 = c0, c1
        hs.append((h0, h1))
        ws.append((jnp.float32(1.0) - w, w))
    return hs, ws


def _rows_of(e, l):
    row = lax.shift_right_logical(e, 2) + jnp.int32(l * _LROWS)
    return row, lax.shift_left(e & jnp.int32(3), 1)


def _rows_of_nat(e, l):
    r = lax.shift_right_logical(e, 3)
    row0 = r + (r & jnp.int32(~15)) + jnp.int32(l * _LROWS)
    return row0, e & jnp.int32(7)


def _ileave_body(t, ti, land, stage):
    wid = lax.axis_index("s") * _NC + lax.axis_index("c")
    lane = lax.iota(jnp.int32, 16)
    l3 = lax.shift_right_logical(lane, 3)
    l2 = lax.shift_right_logical(lane, 2)
    lw = lane & jnp.int32(7)
    sw = lax.shift_left(lane & jnp.int32(3), 1)
    nrw = t.shape[0] // _NW

    def outer(i, c):
        base = wid * nrw + i * 2048
        pltpu.sync_copy(t.at[pl.ds(base, 2048)], land)

        def inner(k, c2):
            kb = k * jnp.int32(256)
            for bb in range(8):
                for j in range(8):
                    g0r = kb + jnp.int32(bb * 32 + j * 2) + l3
                    f0 = plsc.load_gather(land, [g0r, lw])
                    f1 = plsc.load_gather(land, [g0r + jnp.int32(16), lw])
                    sr = kb + jnp.int32(bb * 32 + j * 4) + l2
                    plsc.store_scatter(stage, [sr, sw], f0)
                    plsc.store_scatter(stage, [sr, sw + 1], f1)
            return c2

        lax.fori_loop(0, 8, inner, 0)
        pltpu.sync_copy(stage, ti.at[pl.ds(base, 2048)])
        return c

    lax.fori_loop(0, nrw // 2048, outer, 0)


def _emit_level3(l, xn, idx_ref, lob_ref, w_ref):
    hs, ws = _corner_hashes_weights(xn, _RES3[l])
    hyz = [[hs[1][by] ^ hs[2][bz] for bz in (0, 1)] for by in (0, 1)]
    wxy = [[ws[0][bx] * ws[1][by] for by in (0, 1)] for bx in (0, 1)]
    for k in range(8):
        bx, by, bz = (k >> 2) & 1, (k >> 1) & 1, k & 1
        e = (hs[0][bx] ^ hyz[by][bz]) & jnp.int32(_MASK)
        row0, off = _rows_of(e, l)
        idx_ref[l, pl.ds(k * 16, 16)] = row0
        lob_ref[l, k] = off
        w_ref[l, k] = wxy[bx][by] * ws[2][bz]


def _emit_level4(lrel, lglob, xn, idx_ref, lob_ref, w_ref):
    res = _RES4[lglob]
    rx, ry, rz = int(res[0]), int(res[1]), int(res[2])
    my, mz, mt = rx, rx * ry, rx * ry * rz
    hs, ws = _corner_hashes_weights(xn, res, want_hash=False)
    sy = (hs[1][0] * jnp.int32(my), hs[1][1] * jnp.int32(my))
    sz = (hs[2][0] * jnp.int32(mz), hs[2][1] * jnp.int32(mz))
    st = (hs[3][0] * jnp.int32(mt), hs[3][1] * jnp.int32(mt))
    szt = [[sz[a] + st[b] for b in (0, 1)] for a in (0, 1)]
    wxy = [[ws[0][bx] * ws[1][by] for by in (0, 1)] for bx in (0, 1)]
    wxyz = [[[wxy[bx][by] * ws[2][bz] for bz in (0, 1)] for by in (0, 1)]
            for bx in (0, 1)]
    o4l = jnp.int32(_O4[lglob])
    for p in range(8):
        by, bz, bt = (p >> 2) & 1, (p >> 1) & 1, p & 1
        pid = hs[0][0] + (sy[by] + szt[bz][bt])
        word = pid * jnp.int32(4) + o4l
        idx_ref[lrel, pl.ds(p * 16, 16)] = lax.shift_right_logical(word, 3)
        lob_ref[lrel, p] = word & jnp.int32(7)
        w_ref[lrel, 2 * p] = wxyz[0][by][bz] * ws[3][bt]
        w_ref[lrel, 2 * p + 1] = wxyz[1][by][bz] * ws[3][bt]


def _combine3(rows_ref, rb, lob_ref, w_ref, l, lane):
    acc0 = acc1 = None
    for k in range(8):
        rk = lane + (rb + k * 16)
        off = lob_ref[l, k]
        f0 = plsc.load_gather(rows_ref, [rk, off])
        f1 = plsc.load_gather(rows_ref, [rk, off + 1])
        wk = w_ref[l, k]
        if acc0 is None:
            acc0, acc1 = f0 * wk, f1 * wk
        else:
            acc0, acc1 = acc0 + f0 * wk, acc1 + f1 * wk
    return acc0, acc1


def _combine4(rows_ref, rb, lob_ref, w_ref, lrel, lane):
    acc0 = acc1 = None
    for p in range(8):
        rk = lane + (rb + p * 16)
        off = lob_ref[lrel, p]
        f00 = plsc.load_gather(rows_ref, [rk, off])
        f10 = plsc.load_gather(rows_ref, [rk, off + 1])
        f01 = plsc.load_gather(rows_ref, [rk, off + 2])
        f11 = plsc.load_gather(rows_ref, [rk, off + 3])
        w0 = w_ref[lrel, 2 * p]
        w1 = w_ref[lrel, 2 * p + 1]
        if acc0 is None:
            acc0 = f00 * w0 + f01 * w1
            acc1 = f10 * w0 + f11 * w1
        else:
            acc0 = acc0 + f00 * w0 + f01 * w1
            acc1 = acc1 + f10 * w0 + f11 * w1
    return acc0, acc1


def _build4_body(chunk, t4, grid2, idxb, offb, land, stage, sem):
    wid = lax.axis_index("s") * _NC + lax.axis_index("c")
    lane = lax.iota(jnp.int32, 16)
    for lglob in range(8 * chunk, 8 * chunk + 8):
        res = _RES4[lglob]
        rx, ry, rz = int(res[0]), int(res[1]), int(res[2])
        pmax = jnp.int32(_P4[lglob] - 1)
        nsb = _NSB4[lglob]
        base_row = _O4[lglob] // 8

        def body_sb(i, carry, lglob=lglob, rx=rx, ry=ry, rz=rz,
                    pmax=pmax, nsb=nsb, base_row=base_row):
            sb = i * _NW + wid

            @pl.when(sb < nsb)
            def _():
                s = sb * _SB
                def idiv(v, d):
                    return ((v.astype(jnp.float32) + jnp.float32(0.5))
                            * jnp.float32(1.0 / d)).astype(jnp.int32)

                for j in range(8):
                    v = jnp.minimum(s + (j * 16) + lane, pmax)
                    q = idiv(v, rx)
                    cx = v - q * jnp.int32(rx)
                    q2 = idiv(q, ry)
                    cy = q - q2 * jnp.int32(ry)
                    ct = idiv(q2, rz)
                    cz = q2 - ct * jnp.int32(rz)
                    e = (cx ^ (cy * jnp.int32(_P[1]))
                         ^ (cz * jnp.int32(_P[2]))
                         ^ (ct * jnp.int32(_P[3]))) & jnp.int32(_MASK)
                    row0, off = _rows_of_nat(e, lglob)
                    idxb[0, pl.ds(j * 16, 16)] = row0
                    idxb[1, pl.ds(j * 16, 16)] = row0 + 16
                    offb[j] = off
                cp0 = pltpu.async_copy(t4.at[idxb.at[0]],
                                       land.at[pl.ds(0, 128)], sem)
                cp1 = pltpu.async_copy(t4.at[idxb.at[1]],
                                       land.at[pl.ds(128, 128)], sem)
                cp0.wait()
                cp1.wait()
                for j in range(8):
                    off = offb[j]
                    f0 = plsc.load_gather(land, [lane + j * 16, off])
                    f1 = plsc.load_gather(land, [lane + 128 + j * 16, off])
                    rel = lane * jnp.int32(4) + jnp.int32(j * 64)
                    hi = lax.shift_right_logical(rel, 3)
                    lo = rel & jnp.int32(7)
                    if j < 7:
                        plsc.store_scatter(stage, [hi, lo], f0)
                        plsc.store_scatter(stage, [hi, lo + 1], f1)
                    reld = rel - 2
                    hid = lax.shift_right_logical(reld, 3)
                    lod = reld & jnp.int32(7)
                    if j == 0:
                        m = lane > 0
                        plsc.store_scatter(stage, [hid, lod], f0, mask=m)
                        plsc.store_scatter(stage, [hid, lod + 1], f1, mask=m)
                    elif j < 7:
                        plsc.store_scatter(stage, [hid, lod], f0)
                        plsc.store_scatter(stage, [hid, lod + 1], f1)
                    else:
                        m = lane == 0
                        plsc.store_scatter(stage, [hid, lod], f0, mask=m)
                        plsc.store_scatter(stage, [hid, lod + 1], f1, mask=m)
                pltpu.sync_copy(
                    stage, grid2.at[pl.ds(base_row + sb * (_SB // 2), _SB // 2)])
            return carry

        lax.fori_loop(0, -(-nsb // _NW), body_sb, 0)


def _dash4d_body(xw, yw, zw, tw, t3, g40, g41, g42, g43, out3, out4,
                 cx, cy, cz, ct, idx3, lob3, w3,
                 idx4, lob4, w4, rows, ob3, ob4, sem):
    grids = (g40, g41, g42, g43)
    wid = lax.axis_index("s") * _NC + lax.axis_index("c")
    base = wid * _CHUNK
    pltpu.sync_copy(xw.at[pl.ds(base, _CHUNK)], cx)
    pltpu.sync_copy(yw.at[pl.ds(base, _CHUNK)], cy)
    pltpu.sync_copy(zw.at[pl.ds(base, _CHUNK)], cz)
    pltpu.sync_copy(tw.at[pl.ds(base, _CHUNK)], ct)

    lane = lax.iota(jnp.int32, 16)
    den = jnp.float32(2.0 * _BOUND)

    def _coords(g, with_t):
        sl = pl.ds(g * 16, 16)
        vals = [cx[sl], cy[sl], cz[sl]] + ([ct[sl]] if with_t else [])
        return [jnp.minimum(jnp.maximum((v + jnp.float32(_BOUND)) / den,
                                        jnp.float32(0.0)), jnp.float32(1.0))
                for v in vals]

    def body(g, carry):
        xn = _coords(g, True)
        cps = []
        for l in range(16):
            _emit_level3(l, xn[:3], idx3, lob3, w3)
            cps.append(pltpu.async_copy(
                t3.at[idx3.at[l]], rows.at[pl.ds(l * 128, 128)], sem))
        for l in range(32):
            _emit_level4(l, l, xn, idx4, lob4, w4)
            gref = grids[l // 8]
            cps.append(pltpu.async_copy(
                gref.at[idx4.at[l]], rows.at[pl.ds(2048 + l * 128, 128)], sem))
        for l in range(16):
            cps[l].wait()
            a0, a1 = _combine3(rows, l * 128, lob3, w3, l, lane)
            plsc.store_scatter(ob3, [lane, jnp.full((16,), 2 * l, jnp.int32)], a0)
            plsc.store_scatter(ob3, [lane, jnp.full((16,), 2 * l + 1, jnp.int32)], a1)
        cp3 = pltpu.async_copy(ob3, out3.at[pl.ds(base + g * 16, 16)], sem)
        for l in range(32):
            cps[16 + l].wait()
            a0, a1 = _combine4(rows, 2048 + l * 128, lob4, w4, l, lane)
            plsc.store_scatter(ob4, [lane, jnp.full((16,), 2 * l, jnp.int32)], a0)
            plsc.store_scatter(ob4, [lane, jnp.full((16,), 2 * l + 1, jnp.int32)], a1)
        cp3.wait()
        pltpu.sync_copy(ob4, out4.at[pl.ds(base + g * 16, 16)])
        return carry

    lax.fori_loop(0, _NG, body, 0)


def kernel(xyzt, table3, table4):
    f32 = jnp.float32
    xw = xyzt[:, 0]
    yw = xyzt[:, 1]
    zw = xyzt[:, 2]
    tw = xyzt[:, 3]
    t3 = (table3.reshape(16, _T // 128, 128, 2).transpose(0, 1, 3, 2)
          .reshape(16 * _LROWS, 8))
    t4 = (table4.reshape(32, _T // 128, 128, 2).transpose(0, 1, 3, 2)
          .reshape(32 * _LROWS, 8))

    mesh = plsc.VectorSubcoreMesh(core_axis_name="c", subcore_axis_name="s",
                                  num_cores=_NC, num_subcores=_NS)
    cparams = pltpu.CompilerParams(
        needs_layout_passes=False, use_tc_tiling_on_sc=False)

    import functools
    ileave = pl.kernel(
        _ileave_body,
        out_type=(jax.ShapeDtypeStruct((16 * _LROWS, 8), f32),),
        mesh=mesh,
        compiler_params=cparams,
        scratch_types=[
            pltpu.VMEM((2048, 8), f32),
            pltpu.VMEM((2048, 8), f32),
        ],
        name="ileave3",
    )
    (t3i,) = ileave(t3)
    grids = []
    for c in range(4):
        build = pl.kernel(
            functools.partial(_build4_body, c),
            out_type=(jax.ShapeDtypeStruct((_G4ROWS[c], 8), f32),),
            mesh=mesh,
            compiler_params=cparams,
            scratch_types=[
                pltpu.VMEM((2, 128), jnp.int32),
                pltpu.VMEM((8, 16), jnp.int32),
                pltpu.VMEM((256, 8), f32),
                pltpu.VMEM((_SB // 2, 8), f32),
                pltpu.SemaphoreType.DMA,
            ],
            name=f"build4_{c}",
        )
        (g,) = build(t4)
        grids.append(g)

    fn = pl.kernel(
        _dash4d_body,
        out_type=(
            jax.ShapeDtypeStruct((_N, 32), f32),
            jax.ShapeDtypeStruct((_N, 64), f32),
        ),
        mesh=mesh,
        compiler_params=cparams,
        scratch_types=[
            pltpu.VMEM((_CHUNK,), f32),
            pltpu.VMEM((_CHUNK,), f32),
            pltpu.VMEM((_CHUNK,), f32),
            pltpu.VMEM((_CHUNK,), f32),
            pltpu.VMEM((16, 128), jnp.int32),
            pltpu.VMEM((16, 8, 16), jnp.int32),
            pltpu.VMEM((16, 8, 16), f32),
            pltpu.VMEM((32, 128), jnp.int32),
            pltpu.VMEM((32, 8, 16), jnp.int32),
            pltpu.VMEM((32, 16, 16), f32),
            pltpu.VMEM((48 * 128, 8), f32),
            pltpu.VMEM((16, 32), f32),
            pltpu.VMEM((16, 64), f32),
            pltpu.SemaphoreType.DMA,
        ],
    )
    out3, out4 = fn(xw, yw, zw, tw, t3i, *grids)
    return out3, out4
